# Initial kernel scaffold; baseline (speedup 1.0000x reference)
import jax
import jax.numpy as jnp
from jax.experimental import pallas as pl


def kernel(input_ids, weight):
    raise NotImplementedError("write your pallas kernel here")


# speedup vs baseline: 1.5724x; 1.5724x over previous
import functools

import jax
import jax.numpy as jnp
from jax import lax
from jax.experimental import pallas as pl
from jax.experimental.pallas import tpu as pltpu
from jax.experimental.pallas import tpu_sc as plsc

NUM_EMB = 1000000
DIM = 64
TOTAL = 16384 * 50

NC = 2
NS = 16
NW = NC * NS

N_PER_W = TOTAL // NW
CHUNK = 128
N_CHUNKS = N_PER_W // CHUNK


def _emb_kernel(ids_hbm, w_hbm, out_hbm, idx_v, rows_v, sem):
    wid = lax.axis_index("s") * NC + lax.axis_index("c")
    base = wid * N_PER_W

    @pl.loop(0, N_CHUNKS)
    def _chunk(j):
        off = base + j * CHUNK
        pltpu.sync_copy(ids_hbm.at[pl.ds(off, CHUNK)], idx_v)
        pltpu.async_copy(w_hbm.at[idx_v], rows_v, sem).wait()
        pltpu.sync_copy(rows_v, out_hbm.at[pl.ds(off, CHUNK)])


@jax.jit
def _emb(flat_ids, weight):
    mesh = plsc.VectorSubcoreMesh(
        core_axis_name="c", subcore_axis_name="s", num_cores=NC, num_subcores=NS
    )
    run = pl.kernel(
        _emb_kernel,
        out_type=jax.ShapeDtypeStruct((TOTAL, DIM), jnp.float32),
        mesh=mesh,
        scratch_types=[
            pltpu.VMEM((CHUNK,), jnp.int32),
            pltpu.VMEM((CHUNK, DIM), jnp.float32),
            pltpu.SemaphoreType.DMA,
        ],
        compiler_params=pltpu.CompilerParams(use_tc_tiling_on_sc=False),
    )
    return run(flat_ids, weight)


def kernel(input_ids, weight):
    b, h = input_ids.shape
    out = _emb(input_ids.reshape(-1), weight)
    return out.reshape(b, h, DIM)


# speedup vs baseline: 1.8509x; 1.1771x over previous
import jax
import jax.numpy as jnp
from jax import lax
from jax.experimental import pallas as pl
from jax.experimental.pallas import tpu as pltpu
from jax.experimental.pallas import tpu_sc as plsc

NUM_EMB = 1000000
DIM = 64
TOTAL = 16384 * 50

NC = 2
NS = 16
NW = NC * NS

N_PER_W = TOTAL // NW
CHUNK = 128
N_CHUNKS = N_PER_W // CHUNK


def _emb_kernel(ids_hbm, w_hbm, out_hbm, idx_v, rows0, rows1, isem,
                gsem0, gsem1, ssem0, ssem1):
    wid = lax.axis_index("s") * NC + lax.axis_index("c")
    base = wid * N_PER_W
    rows = (rows0, rows1)
    gsem = (gsem0, gsem1)
    ssem = (ssem0, ssem1)

    pltpu.async_copy(ids_hbm.at[wid], idx_v, isem).wait()

    def start_gather(j, b):
        pltpu.async_copy(w_hbm.at[idx_v.at[j]], rows[b], gsem[b])

    def wait_gather(j, b):
        pltpu.make_async_copy(w_hbm.at[idx_v.at[j]], rows[b], gsem[b]).wait()

    def start_store(j, b):
        pltpu.async_copy(rows[b], out_hbm.at[pl.ds(base + j * CHUNK, CHUNK)],
                         ssem[b])

    def wait_store(j, b):
        pltpu.make_async_copy(rows[b], out_hbm.at[pl.ds(base + j * CHUNK, CHUNK)],
                              ssem[b]).wait()

    start_gather(0, 0)
    start_gather(1, 1)
    wait_gather(0, 0)
    start_store(0, 0)

    @pl.loop(1, N_CHUNKS - 1, step=2)
    def _body(j0):
        for d in (0, 1):
            j = j0 + d
            b = (1 + d) % 2
            wait_store(j - 1, 1 - b)
            start_gather(j + 1, 1 - b)
            wait_gather(j, b)
            start_store(j, b)

    j = N_CHUNKS - 1
    wait_store(j - 1, 0)
    wait_gather(j, 1)
    start_store(j, 1)
    wait_store(j, 1)


@jax.jit
def _emb(ids3d, weight):
    mesh = plsc.VectorSubcoreMesh(
        core_axis_name="c", subcore_axis_name="s", num_cores=NC, num_subcores=NS
    )
    run = pl.kernel(
        _emb_kernel,
        out_type=jax.ShapeDtypeStruct((TOTAL, DIM), jnp.float32),
        mesh=mesh,
        scratch_types=[
            pltpu.VMEM((N_CHUNKS, CHUNK), jnp.int32),
            pltpu.VMEM((CHUNK, DIM), jnp.float32),
            pltpu.VMEM((CHUNK, DIM), jnp.float32),
            pltpu.SemaphoreType.DMA,
            pltpu.SemaphoreType.DMA,
            pltpu.SemaphoreType.DMA,
            pltpu.SemaphoreType.DMA,
            pltpu.SemaphoreType.DMA,
        ],
        compiler_params=pltpu.CompilerParams(use_tc_tiling_on_sc=False),
    )
    return run(ids3d, weight)


def kernel(input_ids, weight):
    b, h = input_ids.shape
    ids3d = input_ids.reshape(NW, N_CHUNKS, CHUNK)
    out = _emb(ids3d, weight)
    return out.reshape(b, h, DIM)


# speedup vs baseline: 1.8753x; 1.0132x over previous
import jax
import jax.numpy as jnp
from jax import lax
from jax.experimental import pallas as pl
from jax.experimental.pallas import tpu as pltpu
from jax.experimental.pallas import tpu_sc as plsc

NUM_EMB = 1000000
DIM = 64
TOTAL = 16384 * 50

NC = 2
NS = 16
NW = NC * NS

N_PER_W = TOTAL // NW
CHUNK = 128
N_CHUNKS = N_PER_W // CHUNK

NBUF = 6
GDEPTH = 4
SDEPTH = NBUF - GDEPTH


def _emb_kernel(ids_hbm, w_hbm, out_hbm, idx_v, rows, isem, gsem, ssem):
    wid = lax.axis_index("s") * NC + lax.axis_index("c")
    base = wid * N_PER_W

    pltpu.async_copy(ids_hbm.at[wid], idx_v, isem).wait()

    def start_gather(j, b):
        pltpu.async_copy(w_hbm.at[idx_v.at[j]], rows[b], gsem[b])

    def wait_gather(j, b):
        pltpu.make_async_copy(w_hbm.at[idx_v.at[j]], rows[b], gsem[b]).wait()

    def start_store(j, b):
        pltpu.async_copy(rows[b], out_hbm.at[pl.ds(base + j * CHUNK, CHUNK)],
                         ssem[b])

    def wait_store(j, b):
        pltpu.make_async_copy(rows[b], out_hbm.at[pl.ds(base + j * CHUNK, CHUNK)],
                              ssem[b]).wait()

    def body(j, b, do_wait_store, do_gather):
        if do_wait_store:
            wait_store(j - SDEPTH, (b - SDEPTH) % NBUF)
        if do_gather:
            start_gather(j + GDEPTH, (b + GDEPTH) % NBUF)
        wait_gather(j, b)
        start_store(j, b)

    for k in range(GDEPTH):
        start_gather(k, k)

    steady_lo = ((SDEPTH + NBUF - 1) // NBUF) * NBUF
    steady_hi = ((N_CHUNKS - GDEPTH - NBUF) // NBUF) * NBUF + NBUF

    for j in range(steady_lo):
        body(j, j % NBUF, j >= SDEPTH, j + GDEPTH < N_CHUNKS)

    @pl.loop(steady_lo, steady_hi, step=NBUF)
    def _steady(j0):
        for d in range(NBUF):
            body(j0 + d, d, True, True)

    for j in range(steady_hi, N_CHUNKS):
        body(j, j % NBUF, True, j + GDEPTH < N_CHUNKS)

    for j in range(N_CHUNKS - SDEPTH, N_CHUNKS):
        wait_store(j, j % NBUF)


@jax.jit
def _emb(ids3d, weight):
    mesh = plsc.VectorSubcoreMesh(
        core_axis_name="c", subcore_axis_name="s", num_cores=NC, num_subcores=NS
    )

    def kern(ids_hbm, w_hbm, out_hbm, idx_v, *rest):
        rows = rest[:NBUF]
        isem = rest[NBUF]
        gsem = rest[NBUF + 1: 2 * NBUF + 1]
        ssem = rest[2 * NBUF + 1: 3 * NBUF + 1]
        _emb_kernel(ids_hbm, w_hbm, out_hbm, idx_v, rows, isem, gsem, ssem)

    run = pl.kernel(
        kern,
        out_type=jax.ShapeDtypeStruct((TOTAL, DIM), jnp.float32),
        mesh=mesh,
        scratch_types=(
            [pltpu.VMEM((N_CHUNKS, CHUNK), jnp.int32)]
            + [pltpu.VMEM((CHUNK, DIM), jnp.float32)] * NBUF
            + [pltpu.SemaphoreType.DMA] * (2 * NBUF + 1)
        ),
        compiler_params=pltpu.CompilerParams(use_tc_tiling_on_sc=False),
    )
    return run(ids3d, weight)


def kernel(input_ids, weight):
    b, h = input_ids.shape
    ids3d = input_ids.reshape(NW, N_CHUNKS, CHUNK)
    out = _emb(ids3d, weight)
    return out.reshape(b, h, DIM)
